# Initial kernel scaffold; baseline (speedup 1.0000x reference)
import jax
import jax.numpy as jnp
from jax.experimental import pallas as pl


def kernel(path, W_ent, W_rel):
    raise NotImplementedError("write your pallas kernel here")


# speedup vs baseline: 4.1492x; 4.1492x over previous
import jax
import jax.numpy as jnp
from jax import lax
from jax.experimental import pallas as pl
from jax.experimental.pallas import tpu as pltpu
from jax.experimental.pallas import tpu_sc as plsc

_L = 16385
_D = 64
_NREL = 1000
_CHUNK = 128
_NW = 32
_CPW = 4
_ROWS_PW = _CHUNK * _CPW
_PAD = _NW * _ROWS_PW + _CHUNK


def _sc_body(path_hbm, table_hbm, out_hbm, pbuf, cidx, rows, sem):
    nc = 2
    wid = lax.axis_index("s") * nc + lax.axis_index("c")
    off = (lax.iota(jnp.int32, 16) & 1) * _NREL

    base = wid * _ROWS_PW
    pltpu.sync_copy(path_hbm.at[pl.ds(base, _ROWS_PW)], pbuf)
    for j in range(_CPW):
        cj = cidx.at[j]
        for k in range(_CHUNK // 16):
            s = pl.ds(j * _CHUNK + k * 16, 16)
            cj[pl.ds(k * 16, 16)] = pbuf[s] + off
    copies = [
        pltpu.async_copy(
            table_hbm.at[cidx.at[j]],
            rows.at[pl.ds(j * _CHUNK, _CHUNK)],
            sem,
        )
        for j in range(_CPW)
    ]
    for cp in copies:
        cp.wait()
    pltpu.sync_copy(rows, out_hbm.at[pl.ds(base, _ROWS_PW)])

    @pl.when(wid == 0)
    def _():
        tbase = _NW * _ROWS_PW
        pltpu.sync_copy(path_hbm.at[pl.ds(tbase, _CHUNK)],
                        pbuf.at[pl.ds(0, _CHUNK)])
        cj = cidx.at[0]
        for k in range(_CHUNK // 16):
            s = pl.ds(k * 16, 16)
            cj[s] = pbuf[s] + off
        pltpu.async_copy(
            table_hbm.at[cidx.at[0]],
            rows.at[pl.ds(0, _CHUNK)],
            sem,
        ).wait()
        pltpu.sync_copy(rows.at[pl.ds(0, _CHUNK)],
                        out_hbm.at[pl.ds(tbase, _CHUNK)])


def kernel(path, W_ent, W_rel):
    table = jnp.concatenate([W_ent[:_NREL], W_rel[:_NREL]], axis=0)
    p = jnp.zeros((_PAD,), jnp.int32).at[:_L].set(path.astype(jnp.int32))
    mesh = plsc.VectorSubcoreMesh(core_axis_name="c", subcore_axis_name="s")
    out = pl.kernel(
        _sc_body,
        mesh=mesh,
        compiler_params=pltpu.CompilerParams(use_tc_tiling_on_sc=False),
        out_type=jax.ShapeDtypeStruct((_PAD, _D), jnp.float32),
        scratch_types=[
            pltpu.VMEM((_ROWS_PW,), jnp.int32),
            pltpu.VMEM((_CPW, _CHUNK), jnp.int32),
            pltpu.VMEM((_ROWS_PW, _D), jnp.float32),
            pltpu.SemaphoreType.DMA,
        ],
    )(p, table)
    return out[:_L]


# speedup vs baseline: 4.9674x; 1.1972x over previous
import jax
import jax.numpy as jnp
from jax import lax
from jax.experimental import pallas as pl
from jax.experimental.pallas import tpu as pltpu
from jax.experimental.pallas import tpu_sc as plsc

_L = 16385
_D = 64
_NREL = 1000
_CHUNK = 128
_NW = 32
_CPW = 4
_ROWS_PW = _CHUNK * _CPW
_MAIN = _NW * _ROWS_PW
_PPAD = _MAIN + 16


def _sc_body(path_hbm, table_hbm, out_hbm, pbuf, cidx, rows, tidx, trows,
             sem_g, sem_w):
    nc = 2
    wid = lax.axis_index("s") * nc + lax.axis_index("c")
    off = (lax.iota(jnp.int32, 16) & 1) * _NREL

    base = wid * _ROWS_PW
    pltpu.sync_copy(path_hbm.at[pl.ds(base, _ROWS_PW)], pbuf)
    for j in range(_CPW):
        cj = cidx.at[j]
        for k in range(_CHUNK // 16):
            cj[pl.ds(k * 16, 16)] = pbuf[pl.ds(j * _CHUNK + k * 16, 16)] + off
    gathers = [
        pltpu.async_copy(
            table_hbm.at[cidx.at[j]],
            rows.at[pl.ds(j * _CHUNK, _CHUNK)],
            sem_g,
        )
        for j in range(_CPW)
    ]
    writes = []
    for j in range(_CPW):
        gathers[j].wait()
        writes.append(
            pltpu.async_copy(
                rows.at[pl.ds(j * _CHUNK, _CHUNK)],
                out_hbm.at[pl.ds(base + j * _CHUNK, _CHUNK)],
                sem_w,
            )
        )

    @pl.when(wid == 0)
    def _():
        pltpu.sync_copy(path_hbm.at[pl.ds(_MAIN, 16)], tidx)
        tidx[...] = tidx[...] + off
        pltpu.async_copy(table_hbm.at[tidx], trows, sem_g).wait()
        pltpu.async_copy(
            trows.at[pl.ds(0, 1)], out_hbm.at[pl.ds(_MAIN, 1)], sem_w
        ).wait()

    for w in writes:
        w.wait()


def kernel(path, W_ent, W_rel):
    table = jnp.concatenate([W_ent[:_NREL], W_rel[:_NREL]], axis=0)
    p = jnp.zeros((_PPAD,), jnp.int32).at[:_L].set(path.astype(jnp.int32))
    mesh = plsc.VectorSubcoreMesh(core_axis_name="c", subcore_axis_name="s")
    out = pl.kernel(
        _sc_body,
        mesh=mesh,
        compiler_params=pltpu.CompilerParams(use_tc_tiling_on_sc=False),
        out_type=jax.ShapeDtypeStruct((_L, _D), jnp.float32),
        scratch_types=[
            pltpu.VMEM((_ROWS_PW,), jnp.int32),
            pltpu.VMEM((_CPW, _CHUNK), jnp.int32),
            pltpu.VMEM((_ROWS_PW, _D), jnp.float32),
            pltpu.VMEM((16,), jnp.int32),
            pltpu.VMEM((16, _D), jnp.float32),
            pltpu.SemaphoreType.DMA,
            pltpu.SemaphoreType.DMA,
        ],
    )(p, table)
    return out
